# Initial kernel scaffold; baseline (speedup 1.0000x reference)
import jax
import jax.numpy as jnp
from jax.experimental import pallas as pl


def kernel(x, edge_index, edge_attr, batch, We0, W0, gamma0, beta0, We1, W1, gamma1, beta1, We2, W2, gamma2, beta2):
    raise NotImplementedError("write your pallas kernel here")


# speedup vs baseline: 2.4826x; 2.4826x over previous
import functools

import jax
import jax.numpy as jnp
from jax import lax
from jax.experimental import pallas as pl
from jax.experimental.pallas import tpu as pltpu
from jax.experimental.pallas import tpu_sc as plsc

N = 10000
E = 320000
D = 128
DE = 16
EPS = 1e-5

NC = 2
NS = 16
NW = NC * NS
EP = E // NW
C = 80
NCHUNK = EP // C
NPAD = 10240
ROWS_PER_TILE = NPAD // NS
ZR = 128

_EB = 3200


def _emb_body(ea_ref, we0_ref, we1_ref, we2_ref, e0_ref, e1_ref, e2_ref):
    ea = ea_ref[...]
    e0_ref[...] = jnp.dot(ea, we0_ref[...], preferred_element_type=jnp.float32)
    e1_ref[...] = jnp.dot(ea, we1_ref[...], preferred_element_type=jnp.float32)
    e2_ref[...] = jnp.dot(ea, we2_ref[...], preferred_element_type=jnp.float32)


def _edge_embeddings(edge_attr, We0, We1, We2):
    grid = (E // _EB,)
    eb_spec = pl.BlockSpec((_EB, DE), lambda i: (i, 0))
    w_spec = pl.BlockSpec((DE, D), lambda i: (0, 0))
    out_spec = pl.BlockSpec((_EB, D), lambda i: (i, 0))
    return pl.pallas_call(
        _emb_body,
        grid=grid,
        in_specs=[eb_spec, w_spec, w_spec, w_spec],
        out_specs=[out_spec, out_spec, out_spec],
        out_shape=[jax.ShapeDtypeStruct((E, D), jnp.float32)] * 3,
    )(edge_attr, We0, We1, We2)


_sc_mesh = plsc.VectorSubcoreMesh(core_axis_name="c", subcore_axis_name="s")


@functools.partial(
    pl.kernel,
    out_type=jax.ShapeDtypeStruct((NC, NPAD, D), jnp.float32),
    mesh=_sc_mesh,
    scratch_types=[
        pltpu.VMEM_SHARED((NPAD, D), jnp.float32),
        pltpu.VMEM((C,), jnp.int32),
        pltpu.VMEM((C,), jnp.int32),
        pltpu.VMEM((C, D), jnp.float32),
        pltpu.VMEM((C, D), jnp.float32),
        pltpu.VMEM((ZR, D), jnp.float32),
        pltpu.SemaphoreType.DMA,
    ],
)
def _sc_aggregate(x_hbm, src_hbm, dst_hbm, emb_hbm, out_hbm,
                  agg, srcv, dstv, xg, ev, zb, sem):
    cid = lax.axis_index("c")
    sid = lax.axis_index("s")
    wid = sid * NC + cid

    def _zero_row(i, _):
        for j in range(D // 16):
            zb[i, pl.ds(j * 16, 16)] = jnp.zeros((16,), jnp.float32)
        return 0

    lax.fori_loop(0, ZR, _zero_row, 0)
    for z in range(ROWS_PER_TILE // ZR):
        pltpu.sync_copy(zb, agg.at[pl.ds(sid * ROWS_PER_TILE + z * ZR, ZR), :])
    plsc.subcore_barrier()

    def _chunk(k, _):
        base = wid * EP + k * C
        pltpu.sync_copy(src_hbm.at[pl.ds(base, C)], srcv)
        pltpu.sync_copy(dst_hbm.at[pl.ds(base, C)], dstv)
        pltpu.sync_copy(emb_hbm.at[pl.ds(base, C), :], ev)
        pltpu.async_copy(x_hbm.at[srcv], xg, sem).wait()

        def _row(i, _):
            for j in range(D // 16):
                sl = pl.ds(j * 16, 16)
                xg[i, sl] = jnp.maximum(xg[i, sl] + ev[i, sl], 0.0)
            return 0

        lax.fori_loop(0, C, _row, 0)
        pltpu.sync_copy(xg, agg.at[dstv], add=True)
        return 0

    lax.fori_loop(0, NCHUNK, _chunk, 0)
    plsc.subcore_barrier()

    r0 = sid * ROWS_PER_TILE
    pltpu.sync_copy(agg.at[pl.ds(r0, ROWS_PER_TILE), :],
                    out_hbm.at[cid, pl.ds(r0, ROWS_PER_TILE), :])


_NB = 1000
_NBLK = N // _NB


def _dense_body(x_ref, p0_ref, p1_ref, w_ref, h_ref, st_ref, sum_ref, sq_ref):
    i = pl.program_id(0)
    y = x_ref[...] + p0_ref[0] + p1_ref[0]
    h = jnp.maximum(jnp.dot(y, w_ref[...], preferred_element_type=jnp.float32), 0.0)
    h_ref[...] = h

    @pl.when(i == 0)
    def _():
        sum_ref[...] = jnp.zeros_like(sum_ref)
        sq_ref[...] = jnp.zeros_like(sq_ref)

    sum_ref[...] += jnp.sum(h, axis=0, keepdims=True)
    sq_ref[...] += jnp.sum(h * h, axis=0, keepdims=True)
    st_ref[0:1, :] = sum_ref[...]
    st_ref[1:2, :] = sq_ref[...]


def _dense(x, partial, W):
    grid = (_NBLK,)
    x_spec = pl.BlockSpec((_NB, D), lambda i: (i, 0))
    p0_spec = pl.BlockSpec((1, _NB, D), lambda i: (0, i, 0))
    p1_spec = pl.BlockSpec((1, _NB, D), lambda i: (1, i, 0))
    w_spec = pl.BlockSpec((D, D), lambda i: (0, 0))
    h_spec = pl.BlockSpec((_NB, D), lambda i: (i, 0))
    st_spec = pl.BlockSpec((2, D), lambda i: (0, 0))
    return pl.pallas_call(
        _dense_body,
        grid=grid,
        in_specs=[x_spec, p0_spec, p1_spec, w_spec],
        out_specs=[h_spec, st_spec],
        out_shape=[
            jax.ShapeDtypeStruct((N, D), jnp.float32),
            jax.ShapeDtypeStruct((2, D), jnp.float32),
        ],
        scratch_shapes=[
            pltpu.VMEM((1, D), jnp.float32),
            pltpu.VMEM((1, D), jnp.float32),
        ],
    )(x, partial, partial, W)


def _bn_body(h_ref, x_ref, st_ref, g_ref, b_ref, o_ref):
    inv_n = jnp.float32(1.0 / N)
    mu = st_ref[0:1, :] * inv_n
    var = st_ref[1:2, :] * inv_n - mu * mu
    inv = lax.rsqrt(var + EPS)
    o_ref[...] = (h_ref[...] - mu) * inv * g_ref[...] + b_ref[...] + x_ref[...]


def _bn_residual(h, x, stats, gamma, beta):
    grid = (_NBLK,)
    blk = pl.BlockSpec((_NB, D), lambda i: (i, 0))
    row = pl.BlockSpec((1, D), lambda i: (0, 0))
    st_spec = pl.BlockSpec((2, D), lambda i: (0, 0))
    return pl.pallas_call(
        _bn_body,
        grid=grid,
        in_specs=[blk, blk, st_spec, row, row],
        out_specs=blk,
        out_shape=jax.ShapeDtypeStruct((N, D), jnp.float32),
    )(h, x, stats, gamma, beta)


def kernel(x, edge_index, edge_attr, batch,
           We0, W0, gamma0, beta0,
           We1, W1, gamma1, beta1,
           We2, W2, gamma2, beta2):
    del batch
    src = edge_index[0]
    dst = edge_index[1]
    embs = _edge_embeddings(edge_attr, We0, We1, We2)
    params = [(W0, gamma0, beta0), (W1, gamma1, beta1), (W2, gamma2, beta2)]
    for emb, (W, g, b) in zip(embs, params):
        partial = _sc_aggregate(x, src, dst, emb)
        h, stats = _dense(x, partial, W)
        x = _bn_residual(h, x, stats, g.reshape(1, D), b.reshape(1, D))
    return x
